# baseline (speedup 1.0000x reference)
import functools

import jax
import jax.numpy as jnp
from jax import lax
from jax.experimental import pallas as pl
from jax.experimental.pallas import tpu as pltpu
from jax.experimental.pallas import tpu_sc as plsc

PAD = 0.0
B, L = 16384, 200
W = L + 1
NS = 16
HB = B // 2
RPW = HB // NS
BS = 4096

_CHUNK_OFFS = tuple(range(0, L - 16, 16)) + (L - 16,)


def _make_sc_lengths(half):
    mesh = plsc.VectorSubcoreMesh(
        core_axis_name="c", subcore_axis_name="s", num_cores=1
    )

    @functools.partial(
        pl.kernel,
        mesh=mesh,
        out_type=jax.ShapeDtypeStruct((HB,), jnp.int32),
        scratch_types=[
            pltpu.VMEM((RPW // 2, L), jnp.float32),
            pltpu.VMEM((RPW,), jnp.int32),
        ],
        compiler_params=pltpu.CompilerParams(use_tc_tiling_on_sc=True),
    )
    def sc_lengths(col_hbm, rl_hbm, buf, lens_v):
        wid = lax.axis_index("s")
        base = half * HB + wid * RPW
        iota16 = lax.iota(jnp.int32, 16)

        pos_vecs = [iota16 + (off + 1) for off in _CHUNK_OFFS]
        rot_idx = [(iota16 + s) % 16 for s in (8, 4, 2, 1)]

        ch = RPW // 2
        for c in range(2):
            pltpu.sync_copy(col_hbm.at[pl.ds(base + c * ch, ch), :], buf)

            def group_body(g, carry, _c=c):
                lenvec = jnp.zeros((16,), jnp.int32)
                for rr in range(16):
                    r = g * 16 + rr
                    acc = jnp.zeros((16,), jnp.int32)
                    for off, pos in zip(_CHUNK_OFFS, pos_vecs):
                        x = buf[r, pl.ds(off, 16)]
                        acc = jnp.where(x != PAD, pos, acc)
                    for idx in rot_idx:
                        acc = jnp.maximum(
                            acc, acc.at[idx].get(mode="promise_in_bounds")
                        )
                    lenvec = jnp.where(iota16 == rr, acc + 1, lenvec)
                lens_v[pl.ds(_c * ch + g * 16, 16)] = lenvec
                return carry

            lax.fori_loop(0, ch // 16, group_body, 0)

        pltpu.sync_copy(lens_v, rl_hbm.at[pl.ds(wid * RPW, RPW)])

    return sc_lengths


_sc_lengths_lo = _make_sc_lengths(0)
_sc_lengths_hi = _make_sc_lengths(1)


def _copy_block(x_ref, dense_ref):
    dense_ref[:, :L] = x_ref[...]
    dense_ref[:, L:] = jnp.zeros((BS, 1), jnp.float32)


def _tc_dense(column):
    return pl.pallas_call(
        _copy_block,
        grid=(B // BS,),
        in_specs=[pl.BlockSpec((BS, L), lambda i: (i, 0))],
        out_specs=pl.BlockSpec((BS, W), lambda i: (i, 0)),
        out_shape=jax.ShapeDtypeStruct((B, W), jnp.float32),
    )(column)


@jax.jit
def kernel(column):
    rl_lo = _sc_lengths_lo(column)
    rl_hi = _sc_lengths_hi(column)
    dense = _tc_dense(column)
    row_lengths = jnp.concatenate([rl_lo, rl_hi])
    return dense, row_lengths


# speedup vs baseline: 1.1980x; 1.1980x over previous
import functools

import jax
import jax.numpy as jnp
from jax import lax
from jax.experimental import pallas as pl
from jax.experimental.pallas import tpu as pltpu
from jax.experimental.pallas import tpu_sc as plsc

PAD = 0.0
B, L = 16384, 200
W = L + 1
NW = 32
RPW = B // NW
NG = RPW // 16
BS = 4096

_CHUNK_OFFS = tuple(range(0, L - 16, 16)) + (L - 16,)

_mesh = plsc.VectorSubcoreMesh(core_axis_name="c", subcore_axis_name="s")


@functools.partial(
    pl.kernel,
    mesh=_mesh,
    out_type=jax.ShapeDtypeStruct((B,), jnp.int32),
    scratch_types=[
        pltpu.VMEM((RPW // 2, L), jnp.float32),
        pltpu.VMEM((RPW,), jnp.int32),
    ],
    compiler_params=pltpu.CompilerParams(use_tc_tiling_on_sc=True),
)
def _sc_lengths(col_hbm, rl_hbm, buf, lens_v):
    wid = lax.axis_index("s") * 2 + lax.axis_index("c")
    base = wid * RPW
    iota16 = lax.iota(jnp.int32, 16)

    pos_vecs = [iota16 + (off + 1) for off in _CHUNK_OFFS]
    rot_idx = [(iota16 + s) % 16 for s in (8, 4, 2, 1)]

    ch = RPW // 2
    for c in range(2):
        pltpu.sync_copy(col_hbm.at[pl.ds(base + c * ch, ch), :], buf)

        def group_body(g, carry, _c=c):
            lenvec = jnp.zeros((16,), jnp.int32)
            for rr in range(16):
                r = g * 16 + rr
                acc = jnp.zeros((16,), jnp.int32)
                for off, pos in zip(_CHUNK_OFFS, pos_vecs):
                    x = buf[r, pl.ds(off, 16)]
                    acc = jnp.where(x != PAD, pos, acc)
                for idx in rot_idx:
                    acc = jnp.maximum(
                        acc, acc.at[idx].get(mode="promise_in_bounds")
                    )
                lenvec = jnp.where(iota16 == rr, acc + 1, lenvec)
            lens_v[pl.ds(_c * ch + g * 16, 16)] = lenvec
            return carry

        lax.fori_loop(0, ch // 16, group_body, 0)

    pltpu.sync_copy(lens_v, rl_hbm.at[pl.ds(base, RPW)])


def _copy_block(x_ref, dense_ref):
    dense_ref[:, :L] = x_ref[...]
    dense_ref[:, L:] = jnp.zeros((BS, 1), jnp.float32)


def _tc_dense(column):
    return pl.pallas_call(
        _copy_block,
        grid=(B // BS,),
        in_specs=[pl.BlockSpec((BS, L), lambda i: (i, 0))],
        out_specs=pl.BlockSpec((BS, W), lambda i: (i, 0)),
        out_shape=jax.ShapeDtypeStruct((B, W), jnp.float32),
    )(column)


@jax.jit
def kernel(column):
    row_lengths = _sc_lengths(column)
    dense = _tc_dense(column)
    return dense, row_lengths


# speedup vs baseline: 1.2230x; 1.0208x over previous
import functools

import jax
import jax.numpy as jnp
from jax import lax
from jax.experimental import pallas as pl
from jax.experimental.pallas import tpu as pltpu
from jax.experimental.pallas import tpu_sc as plsc

PAD = 0.0
B, L = 16384, 200
W = L + 1
NW = 32
RPW = B // NW
NG = RPW // 16
BS = 4096

_CHUNK_OFFS = tuple(range(0, L - 16, 16)) + (L - 16,)

_mesh = plsc.VectorSubcoreMesh(core_axis_name="c", subcore_axis_name="s")


@functools.partial(
    pl.kernel,
    mesh=_mesh,
    out_type=jax.ShapeDtypeStruct((B,), jnp.int32),
    scratch_types=[
        pltpu.VMEM((176, L), jnp.float32),
        pltpu.VMEM((176, L), jnp.float32),
        pltpu.VMEM((RPW,), jnp.int32),
        pltpu.SemaphoreType.DMA,
        pltpu.SemaphoreType.DMA,
    ],
    compiler_params=pltpu.CompilerParams(use_tc_tiling_on_sc=True),
)
def _sc_lengths(col_hbm, rl_hbm, buf0, buf1, lens_v, sem0, sem1):
    wid = lax.axis_index("s") * 2 + lax.axis_index("c")
    base = wid * RPW
    iota16 = lax.iota(jnp.int32, 16)

    pos_vecs = [iota16 + (off + 1) for off in _CHUNK_OFFS]
    rot_idx = [(iota16 + s) % 16 for s in (8, 4, 2, 1)]

    chunks = ((0, 176), (176, 176), (352, 160))
    bufs = (buf0, buf1)
    sems = (sem0, sem1)

    def stage(c):
        start, rows = chunks[c]
        return pltpu.async_copy(
            col_hbm.at[pl.ds(base + start, rows), :],
            bufs[c % 2].at[pl.ds(0, rows), :],
            sems[c % 2],
        )

    pending = stage(0)
    for c, (start, rows) in enumerate(chunks):
        pending.wait()
        if c + 1 < len(chunks):
            pending = stage(c + 1)
        buf = bufs[c % 2]

        def group_body(g, carry, _start=start, _buf=buf):
            lenvec = jnp.zeros((16,), jnp.int32)
            for rr in range(16):
                r = g * 16 + rr
                acc = jnp.zeros((16,), jnp.int32)
                for off, pos in zip(_CHUNK_OFFS, pos_vecs):
                    x = _buf[r, pl.ds(off, 16)]
                    acc = jnp.where(x != PAD, pos, acc)
                for idx in rot_idx:
                    acc = jnp.maximum(
                        acc, acc.at[idx].get(mode="promise_in_bounds")
                    )
                lenvec = jnp.where(iota16 == rr, acc + 1, lenvec)
            lens_v[pl.ds(_start + g * 16, 16)] = lenvec
            return carry

        lax.fori_loop(0, rows // 16, group_body, 0)

    pltpu.sync_copy(lens_v, rl_hbm.at[pl.ds(base, RPW)])


def _copy_block(x_ref, dense_ref):
    dense_ref[:, :L] = x_ref[...]
    dense_ref[:, L:] = jnp.zeros((BS, 1), jnp.float32)


def _tc_dense(column):
    return pl.pallas_call(
        _copy_block,
        grid=(B // BS,),
        in_specs=[pl.BlockSpec((BS, L), lambda i: (i, 0))],
        out_specs=pl.BlockSpec((BS, W), lambda i: (i, 0)),
        out_shape=jax.ShapeDtypeStruct((B, W), jnp.float32),
    )(column)


@jax.jit
def kernel(column):
    row_lengths = _sc_lengths(column)
    dense = _tc_dense(column)
    return dense, row_lengths
